# baseline (speedup 1.0000x reference)
import functools

import jax
import jax.numpy as jnp
from jax import lax
from jax.experimental import pallas as pl
from jax.experimental.pallas import tpu as pltpu
from jax.experimental.pallas import tpu_sc as plsc

N_TOKENS = 8192
HIDDEN = 32
BATCH = 4096
BM = 256
KC = 512
NK = N_TOKENS // KC

NW = 32
RPW = BATCH // NW
RPB = 4


def _prep_body(h_ref, ct_ref, h1_ref, a_ref):
    ct = ct_ref[...]
    cn = jnp.sum(ct * ct, axis=0, keepdims=True)
    a_ref[...] = jnp.concatenate([ct * -2.0, cn], axis=0)
    h1_ref[...] = jnp.concatenate(
        [h_ref[...], jnp.ones((BATCH, 1), jnp.float32)], axis=1)


def _tc_body(h1_ref, a_ref, idx_ref, s_ref):
    h1 = h1_ref[...]

    cmin_cols = []
    for k in range(NK):
        scores = jax.lax.dot_general(
            h1, a_ref[:, k * KC:(k + 1) * KC],
            dimension_numbers=(((1,), (0,)), ((), ())),
            preferred_element_type=jnp.float32,
            precision=jax.lax.Precision.HIGHEST,
        )
        s_ref[:, k * KC:(k + 1) * KC] = scores
        cmin_cols.append(jnp.min(scores, axis=1, keepdims=True))

    cmins = jnp.concatenate(cmin_cols, axis=1)
    gmin = jnp.min(cmins, axis=1, keepdims=True)

    run_idx = jnp.full((BM, 1), N_TOKENS, jnp.int32)
    for k in range(NK):
        scores = s_ref[:, k * KC:(k + 1) * KC]
        iota = jax.lax.broadcasted_iota(jnp.int32, (BM, KC), 1) + k * KC
        cidx = jnp.min(jnp.where(scores == gmin, iota, N_TOKENS),
                       axis=1, keepdims=True)
        run_idx = jnp.minimum(run_idx, cidx)
    idx_ref[...] = run_idx


@functools.partial(
    pl.kernel,
    mesh=plsc.VectorSubcoreMesh(core_axis_name="c", subcore_axis_name="s"),
    out_type=jax.ShapeDtypeStruct((BATCH * N_TOKENS,), jnp.float32),
    scratch_types=[
        pltpu.VMEM((RPW,), jnp.int32),
        pltpu.VMEM((RPB * N_TOKENS,), jnp.float32),
        pltpu.VMEM((RPB * N_TOKENS,), jnp.float32),
        pltpu.SemaphoreType.DMA,
        pltpu.SemaphoreType.DMA,
    ],
    compiler_params=pltpu.CompilerParams(needs_layout_passes=False),
)
def _sc_writer(idx_hbm, z_hbm, out_hbm, idx_v, buf_a, buf_b, sem_a, sem_b):
    wid = lax.axis_index("s") * 2 + lax.axis_index("c")
    base = wid * RPW
    pltpu.sync_copy(idx_hbm.at[pl.ds(base, RPW)], idx_v)
    pltpu.sync_copy(z_hbm, buf_a)
    pltpu.sync_copy(z_hbm, buf_b)
    ones = jnp.full((16,), 1.0, jnp.float32)
    zeros = jnp.zeros((16,), jnp.float32)
    lane = lax.iota(jnp.int32, 16)
    rowoff = (lane & (RPB - 1)) * N_TOKENS
    for g in range(RPW // 16):
        idx16 = idx_v[pl.ds(g * 16, 16)]
        pos = rowoff + idx16
        for ph in range(2):
            lo = ph * 2 * RPB
            mask_a = (lane >= lo) & (lane < lo + RPB)
            mask_b = (lane >= lo + RPB) & (lane < lo + 2 * RPB)
            plsc.store_scatter(buf_a, [pos], ones, mask=mask_a)
            plsc.store_scatter(buf_b, [pos], ones, mask=mask_b)
            flat_a = (base + g * 16 + lo) * N_TOKENS
            flat_b = (base + g * 16 + lo + RPB) * N_TOKENS
            ca = pltpu.async_copy(
                buf_a, out_hbm.at[pl.ds(flat_a, RPB * N_TOKENS)], sem_a)
            cb = pltpu.async_copy(
                buf_b, out_hbm.at[pl.ds(flat_b, RPB * N_TOKENS)], sem_b)
            ca.wait()
            cb.wait()
            plsc.store_scatter(buf_a, [pos], zeros, mask=mask_a)
            plsc.store_scatter(buf_b, [pos], zeros, mask=mask_b)


def kernel(h, temperature, codebook):
    del temperature
    ct = codebook.T
    h1, a = pl.pallas_call(
        _prep_body,
        in_specs=[
            pl.BlockSpec((BATCH, HIDDEN), lambda: (0, 0)),
            pl.BlockSpec((HIDDEN, N_TOKENS), lambda: (0, 0)),
        ],
        out_specs=[
            pl.BlockSpec((BATCH, HIDDEN + 1), lambda: (0, 0)),
            pl.BlockSpec((HIDDEN + 1, N_TOKENS), lambda: (0, 0)),
        ],
        out_shape=[
            jax.ShapeDtypeStruct((BATCH, HIDDEN + 1), jnp.float32),
            jax.ShapeDtypeStruct((HIDDEN + 1, N_TOKENS), jnp.float32),
        ],
    )(h, ct)
    idx = pl.pallas_call(
        _tc_body,
        grid=(BATCH // BM,),
        in_specs=[
            pl.BlockSpec((BM, HIDDEN + 1), lambda i: (i, 0)),
            pl.BlockSpec((HIDDEN + 1, N_TOKENS), lambda i: (0, 0)),
        ],
        out_specs=pl.BlockSpec((BM, 1), lambda i: (i, 0)),
        out_shape=jax.ShapeDtypeStruct((BATCH, 1), jnp.int32),
        scratch_shapes=[pltpu.VMEM((BM, N_TOKENS), jnp.float32)],
    )(h1, a)
    zeros_blk = jnp.zeros((RPB * N_TOKENS,), jnp.float32)
    out_flat = _sc_writer(idx.reshape(BATCH), zeros_blk)
    return out_flat.reshape(BATCH, N_TOKENS)


# speedup vs baseline: 2.6880x; 2.6880x over previous
import jax
import jax.numpy as jnp
from jax.experimental import pallas as pl
from jax.experimental.pallas import tpu as pltpu

N_TOKENS = 8192
HIDDEN = 32
BATCH = 4096
BM = 256
KC = 512
NK = N_TOKENS // KC


def _prep_body(h_ref, ct_ref, h1_ref, a_ref):
    ct = ct_ref[...]
    cn = jnp.sum(ct * ct, axis=0, keepdims=True)
    a_ref[...] = jnp.concatenate([ct * -2.0, cn], axis=0)
    h1_ref[...] = jnp.concatenate(
        [h_ref[...], jnp.ones((BATCH, 1), jnp.float32)], axis=1)


def _body(h1_ref, a_ref, o_ref, s_ref):
    h1 = h1_ref[...]
    hh = h1[:, :HIDDEN]
    hn = jnp.sum(hh * hh, axis=1, keepdims=True)

    cmin_cols = []
    for k in range(NK):
        scores = jax.lax.dot_general(
            h1, a_ref[:, k * KC:(k + 1) * KC],
            dimension_numbers=(((1,), (0,)), ((), ())),
            preferred_element_type=jnp.float32,
            precision=jax.lax.Precision.HIGHEST,
        )
        dists = hn + scores
        s_ref[:, k * KC:(k + 1) * KC] = dists
        cmin_cols.append(jnp.min(dists, axis=1, keepdims=True))

    cmins = jnp.concatenate(cmin_cols, axis=1)
    gmin = jnp.min(cmins, axis=1, keepdims=True)
    iota = jax.lax.broadcasted_iota(jnp.int32, cmins.shape, 1)
    kwin = jnp.min(jnp.where(cmins == gmin, iota, NK),
                   axis=1, keepdims=True)

    for k in range(NK):
        scores = s_ref[:, k * KC:(k + 1) * KC]
        tgt = jnp.where(kwin == k, gmin, jnp.inf)
        o_ref[:, k * KC:(k + 1) * KC] = (scores == tgt).astype(jnp.float32)


def kernel(h, temperature, codebook):
    del temperature
    ct = codebook.T
    h1, a = pl.pallas_call(
        _prep_body,
        in_specs=[
            pl.BlockSpec((BATCH, HIDDEN), lambda: (0, 0)),
            pl.BlockSpec((HIDDEN, N_TOKENS), lambda: (0, 0)),
        ],
        out_specs=[
            pl.BlockSpec((BATCH, HIDDEN + 1), lambda: (0, 0)),
            pl.BlockSpec((HIDDEN + 1, N_TOKENS), lambda: (0, 0)),
        ],
        out_shape=[
            jax.ShapeDtypeStruct((BATCH, HIDDEN + 1), jnp.float32),
            jax.ShapeDtypeStruct((HIDDEN + 1, N_TOKENS), jnp.float32),
        ],
    )(h, ct)
    return pl.pallas_call(
        _body,
        grid=(BATCH // BM,),
        in_specs=[
            pl.BlockSpec((BM, HIDDEN + 1), lambda i: (i, 0)),
            pl.BlockSpec((HIDDEN + 1, N_TOKENS), lambda i: (0, 0)),
        ],
        out_specs=pl.BlockSpec((BM, N_TOKENS), lambda i: (i, 0)),
        out_shape=jax.ShapeDtypeStruct((BATCH, N_TOKENS), jnp.float32),
        scratch_shapes=[pltpu.VMEM((BM, N_TOKENS), jnp.float32)],
    )(h1, a)


# speedup vs baseline: 5.7748x; 2.1484x over previous
import jax
import jax.numpy as jnp
from jax.experimental import pallas as pl
from jax.experimental.pallas import tpu as pltpu

N_TOKENS = 8192
HIDDEN = 32
BATCH = 4096
BM = 256
KC = 512
NK = N_TOKENS // KC
K1 = HIDDEN + 1
K6 = 6 * K1


def _split3(x):
    hi = x.astype(jnp.bfloat16)
    mid = (x - hi.astype(jnp.float32)).astype(jnp.bfloat16)
    lo = (x - hi.astype(jnp.float32) - mid.astype(jnp.float32)
          ).astype(jnp.bfloat16)
    return hi, mid, lo


def _prep_body(h_ref, ct_ref, hcat_ref, acat_ref, hn_ref):
    h = h_ref[...]
    hn_ref[...] = jnp.sum(h * h, axis=1, keepdims=True)
    h1 = jnp.concatenate(
        [h, jnp.ones((BATCH, 1), jnp.float32)], axis=1)
    ct = ct_ref[...]
    cn = jnp.sum(ct * ct, axis=0, keepdims=True)
    a = jnp.concatenate([ct * -2.0, cn], axis=0)
    hh, hm, hl = _split3(h1)
    ah, am, al = _split3(a)
    hcat_ref[...] = jnp.concatenate([hl, hm, hh, hm, hh, hh], axis=1)
    acat_ref[...] = jnp.concatenate([ah, am, al, ah, am, ah], axis=0)


def _body(hcat_ref, acat_ref, hn_ref, o_ref, s_ref):
    hcat = hcat_ref[...]
    hn = hn_ref[...]

    cmin_cols = []
    for k in range(NK):
        scores = jax.lax.dot_general(
            hcat, acat_ref[:, k * KC:(k + 1) * KC],
            dimension_numbers=(((1,), (0,)), ((), ())),
            preferred_element_type=jnp.float32,
        )
        dists = hn + scores
        s_ref[:, k * KC:(k + 1) * KC] = dists
        cmin_cols.append(jnp.min(dists, axis=1, keepdims=True))

    cmins = jnp.concatenate(cmin_cols, axis=1)
    gmin = jnp.min(cmins, axis=1, keepdims=True)
    iota = jax.lax.broadcasted_iota(jnp.int32, cmins.shape, 1)
    kwin = jnp.min(jnp.where(cmins == gmin, iota, NK),
                   axis=1, keepdims=True)

    for k in range(NK):
        dists = s_ref[:, k * KC:(k + 1) * KC]
        tgt = jnp.where(kwin == k, gmin, jnp.inf)
        o_ref[:, k * KC:(k + 1) * KC] = (dists == tgt).astype(jnp.float32)


def kernel(h, temperature, codebook):
    del temperature
    ct = codebook.T
    hcat, acat, hn = pl.pallas_call(
        _prep_body,
        in_specs=[
            pl.BlockSpec((BATCH, HIDDEN), lambda: (0, 0)),
            pl.BlockSpec((HIDDEN, N_TOKENS), lambda: (0, 0)),
        ],
        out_specs=[
            pl.BlockSpec((BATCH, K6), lambda: (0, 0)),
            pl.BlockSpec((K6, N_TOKENS), lambda: (0, 0)),
            pl.BlockSpec((BATCH, 1), lambda: (0, 0)),
        ],
        out_shape=[
            jax.ShapeDtypeStruct((BATCH, K6), jnp.bfloat16),
            jax.ShapeDtypeStruct((K6, N_TOKENS), jnp.bfloat16),
            jax.ShapeDtypeStruct((BATCH, 1), jnp.float32),
        ],
    )(h, ct)
    return pl.pallas_call(
        _body,
        grid=(BATCH // BM,),
        in_specs=[
            pl.BlockSpec((BM, K6), lambda i: (i, 0)),
            pl.BlockSpec((K6, N_TOKENS), lambda i: (0, 0)),
            pl.BlockSpec((BM, 1), lambda i: (i, 0)),
        ],
        out_specs=pl.BlockSpec((BM, N_TOKENS), lambda i: (i, 0)),
        out_shape=jax.ShapeDtypeStruct((BATCH, N_TOKENS), jnp.float32),
        scratch_shapes=[pltpu.VMEM((BM, N_TOKENS), jnp.float32)],
    )(hcat, acat, hn)
